# Initial kernel scaffold; baseline (speedup 1.0000x reference)
import jax
import jax.numpy as jnp
from jax.experimental import pallas as pl


def kernel(x, edge_index, W, b):
    raise NotImplementedError("write your pallas kernel here")


# speedup vs baseline: 68.0481x; 68.0481x over previous
import jax
import jax.numpy as jnp
from jax import lax
from jax.experimental import pallas as pl
from jax.experimental.pallas import tpu as pltpu
from jax.experimental.pallas import tpu_sc as plsc

N = 10000
E = 320000
D = 128
H = 4

NC = 2
NS = 16
NW = NC * NS
CH = 128
RPW = 80
EPAD = NW * RPW * CH
NPAD = 8


def _deg_body(dst_hbm, ones_hbm, zeros_hbm, out_hbm, deg_sp, dstv, onesv, zv):
    c = lax.axis_index("c")
    s = lax.axis_index("s")
    wid = c * NS + s
    pltpu.sync_copy(dst_hbm.at[wid], dstv)
    pltpu.sync_copy(ones_hbm, onesv)
    @pl.when(s < 10)
    def _():
        pltpu.sync_copy(zeros_hbm, zv)
        pltpu.sync_copy(zv, deg_sp.at[pl.ds(s * 1000, 1000)])
    plsc.subcore_barrier()

    def body(j, carry):
        pltpu.sync_copy(onesv, deg_sp.at[dstv.at[j]], add=True)
        return carry

    lax.fori_loop(0, RPW, body, 0)
    plsc.subcore_barrier()

    @pl.when(s < 10)
    def _():
        pltpu.sync_copy(deg_sp.at[pl.ds(s * 1000, 1000)], zv)
        pltpu.sync_copy(zv, out_hbm.at[c, s])


def _agg_body(y_hbm, src_hbm, dst_hbm, zeros_hbm, out_hbm,
              agg_sp, dstv, sring, bufs, isems, gsems):
    c = lax.axis_index("c")
    s = lax.axis_index("s")
    wid = c * NS + s
    pltpu.sync_copy(dst_hbm.at[wid], dstv)
    pltpu.sync_copy(zeros_hbm, bufs[0])

    @pl.when(s < 15)
    def _():
        for k in range(5):
            pltpu.sync_copy(bufs[0],
                            agg_sp.at[pl.ds(s * 640 + k * 128, 128)])

    @pl.when(s == 15)
    def _():
        for k in range(3):
            pltpu.sync_copy(bufs[0],
                            agg_sp.at[pl.ds(9600 + k * 128, 128)])
        pltpu.sync_copy(bufs[0].at[pl.ds(0, 16)],
                        agg_sp.at[pl.ds(9984, 16)])
    plsc.subcore_barrier()

    for r in range(3):
        pltpu.async_copy(src_hbm.at[wid, r], sring[r], isems[r])
    pltpu.make_async_copy(src_hbm.at[wid, 0], sring[0], isems[0]).wait()
    pltpu.async_copy(y_hbm.at[sring[0]], bufs[0], gsems[0])

    def group(g, carry):
        for u in range(4):
            j = g * 4 + u
            rn = (u + 1) % 4

            @pl.when(j + 1 < RPW)
            def _():
                pltpu.make_async_copy(src_hbm.at[wid, j + 1], sring[rn],
                                      isems[rn]).wait()
                pltpu.async_copy(y_hbm.at[sring[rn]], bufs[(u + 1) % 2],
                                 gsems[(u + 1) % 2])

            @pl.when(j + 3 < RPW)
            def _():
                pltpu.async_copy(src_hbm.at[wid, j + 3], sring[(u + 3) % 4],
                                 isems[(u + 3) % 4])

            pltpu.make_async_copy(y_hbm.at[sring[u]], bufs[u % 2],
                                  gsems[u % 2]).wait()
            pltpu.sync_copy(bufs[u % 2], agg_sp.at[dstv.at[j]], add=True)
        return carry

    lax.fori_loop(0, RPW // 4, group, 0)
    plsc.subcore_barrier()

    @pl.when(s < 15)
    def _():
        pltpu.sync_copy(agg_sp.at[pl.ds(s * 640, 640)],
                        out_hbm.at[c, pl.ds(s * 640, 640)])

    @pl.when(s == 15)
    def _():
        pltpu.sync_copy(agg_sp.at[pl.ds(9600, 400)],
                        out_hbm.at[c, pl.ds(9600, 400)])


def _scale_body(x_ref, dp_ref, y_ref, dis_ref):
    deg = dp_ref[0] + dp_ref[1] + 1.0
    dis = lax.rsqrt(deg)
    y_ref[...] = x_ref[...] * dis
    dis_ref[...] = dis


def _head_body(ap_ref, y_ref, dis_ref, w_ref, b_ref, o_ref):
    agg = (ap_ref[0] + ap_ref[1] + y_ref[...]) * dis_ref[...]
    for h in range(H):
        o_ref[h] = (
            jnp.dot(agg, w_ref[h], preferred_element_type=jnp.float32)
            + b_ref[h][None, :]
        )


def _sc_mesh():
    return plsc.VectorSubcoreMesh(core_axis_name="c", subcore_axis_name="s")


@jax.jit
def kernel(x, edge_index, W, b):
    npad = EPAD - E
    pad_src = (jnp.arange(npad, dtype=jnp.int32) % 128)
    pad_dst = N + (jnp.arange(npad, dtype=jnp.int32) % NPAD)
    src = jnp.concatenate([edge_index[0], pad_src]).reshape(NW, RPW, CH)
    dst = jnp.concatenate([edge_index[1], pad_dst]).reshape(NW, RPW, CH)

    deg_part = pl.kernel(
        _deg_body,
        out_type=jax.ShapeDtypeStruct((NC, 10, 1000), jnp.float32),
        mesh=_sc_mesh(),
        scratch_types=[
            pltpu.VMEM_SHARED((N + NPAD,), jnp.float32),
            pltpu.VMEM((RPW, CH), jnp.int32),
            pltpu.VMEM((CH,), jnp.float32),
            pltpu.VMEM((1000,), jnp.float32),
        ],
    )(dst, jnp.ones((CH,), jnp.float32), jnp.zeros((1000,), jnp.float32))

    y, dis = pl.pallas_call(
        _scale_body,
        out_shape=(
            jax.ShapeDtypeStruct((N, D), jnp.float32),
            jax.ShapeDtypeStruct((N, 1), jnp.float32),
        ),
    )(x, deg_part.reshape(NC, N, 1))

    agg_part = pl.kernel(
        _agg_body,
        out_type=jax.ShapeDtypeStruct((NC, N, D), jnp.float32),
        mesh=_sc_mesh(),
        scratch_types=[
            pltpu.VMEM_SHARED((N + NPAD, D), jnp.float32),
            pltpu.VMEM((RPW, CH), jnp.int32),
            [pltpu.VMEM((CH,), jnp.int32) for _ in range(4)],
            [pltpu.VMEM((CH, D), jnp.float32) for _ in range(2)],
            [pltpu.SemaphoreType.DMA for _ in range(4)],
            [pltpu.SemaphoreType.DMA for _ in range(2)],
        ],
    )(y, src, dst, jnp.zeros((CH, D), jnp.float32))

    bm = 1000
    out4 = pl.pallas_call(
        _head_body,
        grid=(N // bm,),
        in_specs=[
            pl.BlockSpec((NC, bm, D), lambda i: (0, i, 0)),
            pl.BlockSpec((bm, D), lambda i: (i, 0)),
            pl.BlockSpec((bm, 1), lambda i: (i, 0)),
            pl.BlockSpec((H, D, D), lambda i: (0, 0, 0)),
            pl.BlockSpec((H, D), lambda i: (0, 0)),
        ],
        out_specs=pl.BlockSpec((H, bm, D), lambda i: (0, i, 0)),
        out_shape=jax.ShapeDtypeStruct((H, N, D), jnp.float32),
    )(agg_part, y, dis, W, b)

    return jnp.transpose(out4, (1, 2, 0))


# speedup vs baseline: 69.8789x; 1.0269x over previous
import jax
import jax.numpy as jnp
from jax import lax
from jax.experimental import pallas as pl
from jax.experimental.pallas import tpu as pltpu
from jax.experimental.pallas import tpu_sc as plsc

N = 10000
E = 320000
D = 128
H = 4

NC = 2
NS = 16
NW = NC * NS
CH = 128
TR = E // CH
RPW = 80


def _worker_rows(wid):
    nch = jnp.where(wid == NW - 1, TR - (NW - 1) * RPW, RPW)
    return wid * RPW, nch


def _deg_body(dst_hbm, ones_hbm, zeros_hbm, out_hbm,
              deg_sp, dstv, onesv, zv, dsems):
    c = lax.axis_index("c")
    s = lax.axis_index("s")
    wid = c * NS + s
    base, nch = _worker_rows(wid)

    @pl.when(wid < NW - 1)
    def _():
        pltpu.sync_copy(dst_hbm.at[pl.ds(base, RPW)], dstv)

    @pl.when(wid == NW - 1)
    def _():
        pltpu.sync_copy(dst_hbm.at[pl.ds((NW - 1) * RPW, 20)],
                        dstv.at[pl.ds(0, 20)])
    pltpu.sync_copy(ones_hbm, onesv)
    @pl.when(s < 10)
    def _():
        pltpu.sync_copy(zeros_hbm, zv)
        pltpu.sync_copy(zv, deg_sp.at[pl.ds(s * 1000, 1000)])
    plsc.subcore_barrier()

    def body(g, carry):
        for u in range(4):
            j = g * 4 + u

            @pl.when(j >= 4)
            def _():
                pltpu.make_async_copy(onesv, deg_sp.at[dstv.at[0]],
                                      dsems[u]).wait()
            pltpu.async_copy(onesv, deg_sp.at[dstv.at[j]], dsems[u],
                             add=True)
        return carry

    lax.fori_loop(0, nch // 4, body, 0)
    for u in range(4):
        pltpu.make_async_copy(onesv, deg_sp.at[dstv.at[0]], dsems[u]).wait()
    plsc.subcore_barrier()

    @pl.when(s < 10)
    def _():
        pltpu.sync_copy(deg_sp.at[pl.ds(s * 1000, 1000)], zv)
        pltpu.sync_copy(zv, out_hbm.at[c, s])


def _agg_body(y_hbm, src_hbm, dst_hbm, zeros_hbm, out_hbm,
              agg_sp, dstv, sring, bufs, isems, gsems, ssems):
    c = lax.axis_index("c")
    s = lax.axis_index("s")
    wid = c * NS + s
    base, nch = _worker_rows(wid)

    @pl.when(wid < NW - 1)
    def _():
        pltpu.sync_copy(dst_hbm.at[pl.ds(base, RPW)], dstv)

    @pl.when(wid == NW - 1)
    def _():
        pltpu.sync_copy(dst_hbm.at[pl.ds((NW - 1) * RPW, 20)],
                        dstv.at[pl.ds(0, 20)])
    pltpu.sync_copy(zeros_hbm, bufs[0])

    @pl.when(s < 15)
    def _():
        for k in range(5):
            pltpu.sync_copy(bufs[0],
                            agg_sp.at[pl.ds(s * 640 + k * 128, 128)])

    @pl.when(s == 15)
    def _():
        for k in range(3):
            pltpu.sync_copy(bufs[0],
                            agg_sp.at[pl.ds(9600 + k * 128, 128)])
        pltpu.sync_copy(bufs[0].at[pl.ds(0, 16)],
                        agg_sp.at[pl.ds(9984, 16)])
    plsc.subcore_barrier()

    for r in range(3):
        pltpu.async_copy(src_hbm.at[base + r], sring[r], isems[r])
    pltpu.make_async_copy(src_hbm.at[base], sring[0], isems[0]).wait()
    pltpu.async_copy(y_hbm.at[sring[0]], bufs[0], gsems[0])

    def group(g, carry):
        for u in range(4):
            j = g * 4 + u
            rn = (u + 1) % 4
            bn = (u + 1) % 2

            @pl.when(j + 1 < nch)
            def _():
                @pl.when(j >= 1)
                def _():
                    pltpu.make_async_copy(bufs[bn], agg_sp.at[dstv.at[0]],
                                          ssems[bn]).wait()
                pltpu.make_async_copy(src_hbm.at[base + j + 1], sring[rn],
                                      isems[rn]).wait()
                pltpu.async_copy(y_hbm.at[sring[rn]], bufs[bn], gsems[bn])

            @pl.when(j + 3 < nch)
            def _():
                pltpu.async_copy(src_hbm.at[base + j + 3], sring[(u + 3) % 4],
                                 isems[(u + 3) % 4])

            pltpu.make_async_copy(y_hbm.at[sring[u]], bufs[u % 2],
                                  gsems[u % 2]).wait()
            pltpu.async_copy(bufs[u % 2], agg_sp.at[dstv.at[j]],
                             ssems[u % 2], add=True)
        return carry

    lax.fori_loop(0, nch // 4, group, 0)
    for b in range(2):
        pltpu.make_async_copy(bufs[b], agg_sp.at[dstv.at[0]],
                              ssems[b]).wait()
    plsc.subcore_barrier()

    @pl.when(s < 15)
    def _():
        pltpu.sync_copy(agg_sp.at[pl.ds(s * 640, 640)],
                        out_hbm.at[c, pl.ds(s * 640, 640)])

    @pl.when(s == 15)
    def _():
        pltpu.sync_copy(agg_sp.at[pl.ds(9600, 400)],
                        out_hbm.at[c, pl.ds(9600, 400)])


def _scale_body(x_ref, dp_ref, y_ref, dis_ref):
    deg = dp_ref[0] + dp_ref[1] + 1.0
    dis = lax.rsqrt(deg)
    y_ref[...] = x_ref[...] * dis
    dis_ref[...] = dis


def _head_body(ap_ref, y_ref, dis_ref, w_ref, b_ref, o_ref):
    agg = (ap_ref[0] + ap_ref[1] + y_ref[...]) * dis_ref[...]
    for h in range(H):
        o_ref[h] = (
            jnp.dot(agg, w_ref[h], preferred_element_type=jnp.float32)
            + b_ref[h][None, :]
        )


def _sc_mesh():
    return plsc.VectorSubcoreMesh(core_axis_name="c", subcore_axis_name="s")


@jax.jit
def kernel(x, edge_index, W, b):
    src = edge_index[0].reshape(TR, CH)
    dst = edge_index[1].reshape(TR, CH)

    deg_part = pl.kernel(
        _deg_body,
        out_type=jax.ShapeDtypeStruct((NC, 10, 1000), jnp.float32),
        mesh=_sc_mesh(),
        scratch_types=[
            pltpu.VMEM_SHARED((N,), jnp.float32),
            pltpu.VMEM((RPW, CH), jnp.int32),
            pltpu.VMEM((CH,), jnp.float32),
            pltpu.VMEM((1000,), jnp.float32),
            [pltpu.SemaphoreType.DMA for _ in range(4)],
        ],
    )(dst, jnp.ones((CH,), jnp.float32), jnp.zeros((1000,), jnp.float32))

    y, dis = pl.pallas_call(
        _scale_body,
        out_shape=(
            jax.ShapeDtypeStruct((N, D), jnp.float32),
            jax.ShapeDtypeStruct((N, 1), jnp.float32),
        ),
    )(x, deg_part.reshape(NC, N, 1))

    agg_part = pl.kernel(
        _agg_body,
        out_type=jax.ShapeDtypeStruct((NC, N, D), jnp.float32),
        mesh=_sc_mesh(),
        scratch_types=[
            pltpu.VMEM_SHARED((N, D), jnp.float32),
            pltpu.VMEM((RPW, CH), jnp.int32),
            [pltpu.VMEM((CH,), jnp.int32) for _ in range(4)],
            [pltpu.VMEM((CH, D), jnp.float32) for _ in range(2)],
            [pltpu.SemaphoreType.DMA for _ in range(4)],
            [pltpu.SemaphoreType.DMA for _ in range(2)],
            [pltpu.SemaphoreType.DMA for _ in range(2)],
        ],
    )(y, src, dst, jnp.zeros((CH, D), jnp.float32))

    bm = 1000
    out4 = pl.pallas_call(
        _head_body,
        grid=(N // bm,),
        in_specs=[
            pl.BlockSpec((NC, bm, D), lambda i: (0, i, 0)),
            pl.BlockSpec((bm, D), lambda i: (i, 0)),
            pl.BlockSpec((bm, 1), lambda i: (i, 0)),
            pl.BlockSpec((H, D, D), lambda i: (0, 0, 0)),
            pl.BlockSpec((H, D), lambda i: (0, 0)),
        ],
        out_specs=pl.BlockSpec((H, bm, D), lambda i: (0, i, 0)),
        out_shape=jax.ShapeDtypeStruct((H, N, D), jnp.float32),
    )(agg_part, y, dis, W, b)

    return jnp.transpose(out4, (1, 2, 0))


# speedup vs baseline: 71.4412x; 1.0224x over previous
import jax
import jax.numpy as jnp
from jax import lax
from jax.experimental import pallas as pl
from jax.experimental.pallas import tpu as pltpu
from jax.experimental.pallas import tpu_sc as plsc

N = 10000
E = 320000
D = 128
H = 4

NC = 2
NS = 16
NW = NC * NS
CH = 128
TR = E // CH
RPW = 80


def _worker_rows(wid):
    nch = jnp.where(wid == NW - 1, TR - (NW - 1) * RPW, RPW)
    return wid * RPW, nch


def _ei_chunk(ei_hbm, row):
    return ei_hbm.at[pl.ds(0, 2), pl.ds(row * CH, CH)]


def _deg_body(ei_hbm, ones_hbm, zeros_hbm, out_hbm,
              deg_sp, ring, onesv, zv, isems, dsems):
    c = lax.axis_index("c")
    s = lax.axis_index("s")
    wid = c * NS + s
    base, nch = _worker_rows(wid)
    pltpu.sync_copy(ones_hbm, onesv)
    @pl.when(s < 10)
    def _():
        pltpu.sync_copy(zeros_hbm, zv)
        pltpu.sync_copy(zv, deg_sp.at[pl.ds(s * 1000, 1000)])
    plsc.subcore_barrier()

    for r in range(2):
        pltpu.async_copy(_ei_chunk(ei_hbm, base + r), ring[r], isems[r])

    def body(g, carry):
        for u in range(4):
            j = g * 4 + u

            @pl.when(j >= 2)
            def _():
                pltpu.make_async_copy(onesv, deg_sp.at[ring[0].at[1]],
                                      dsems[u % 2]).wait()

            @pl.when(j + 2 < nch)
            def _():
                pltpu.async_copy(_ei_chunk(ei_hbm, base + j + 2),
                                 ring[(u + 2) % 4], isems[(u + 2) % 4])
            pltpu.make_async_copy(_ei_chunk(ei_hbm, base), ring[u],
                                  isems[u]).wait()
            pltpu.async_copy(onesv, deg_sp.at[ring[u].at[1]], dsems[u % 2],
                             add=True)
        return carry

    lax.fori_loop(0, nch // 4, body, 0)
    for k in range(2):
        pltpu.make_async_copy(onesv, deg_sp.at[ring[0].at[1]],
                              dsems[k]).wait()
    plsc.subcore_barrier()

    @pl.when(s < 10)
    def _():
        pltpu.sync_copy(deg_sp.at[pl.ds(s * 1000, 1000)], zv)
        pltpu.sync_copy(zv, out_hbm.at[c, s])


def _agg_body(y_hbm, ei_hbm, zeros_hbm, out_hbm,
              agg_sp, sring, bufs, isems, gsems, ssems):
    c = lax.axis_index("c")
    s = lax.axis_index("s")
    wid = c * NS + s
    base, nch = _worker_rows(wid)
    pltpu.sync_copy(zeros_hbm, bufs[0])

    @pl.when(s < 15)
    def _():
        for k in range(5):
            pltpu.sync_copy(bufs[0],
                            agg_sp.at[pl.ds(s * 640 + k * 128, 128)])

    @pl.when(s == 15)
    def _():
        for k in range(3):
            pltpu.sync_copy(bufs[0],
                            agg_sp.at[pl.ds(9600 + k * 128, 128)])
        pltpu.sync_copy(bufs[0].at[pl.ds(0, 16)],
                        agg_sp.at[pl.ds(9984, 16)])
    plsc.subcore_barrier()

    for r in range(3):
        pltpu.async_copy(_ei_chunk(ei_hbm, base + r), sring[r], isems[r])
    pltpu.make_async_copy(_ei_chunk(ei_hbm, base), sring[0], isems[0]).wait()
    pltpu.async_copy(y_hbm.at[sring[0].at[0]], bufs[0], gsems[0])

    def group(g, carry):
        for u in range(4):
            j = g * 4 + u
            rn = (u + 1) % 4
            bn = (u + 1) % 2

            @pl.when(j + 1 < nch)
            def _():
                @pl.when(j >= 1)
                def _():
                    pltpu.make_async_copy(bufs[bn],
                                          agg_sp.at[sring[0].at[1]],
                                          ssems[bn]).wait()
                pltpu.make_async_copy(_ei_chunk(ei_hbm, base), sring[rn],
                                      isems[rn]).wait()
                pltpu.async_copy(y_hbm.at[sring[rn].at[0]], bufs[bn],
                                 gsems[bn])

            @pl.when(j + 3 < nch)
            def _():
                pltpu.async_copy(_ei_chunk(ei_hbm, base + j + 3),
                                 sring[(u + 3) % 4], isems[(u + 3) % 4])

            pltpu.make_async_copy(y_hbm.at[sring[u].at[0]], bufs[u % 2],
                                  gsems[u % 2]).wait()
            pltpu.async_copy(bufs[u % 2], agg_sp.at[sring[u].at[1]],
                             ssems[u % 2], add=True)
        return carry

    lax.fori_loop(0, nch // 4, group, 0)
    for b in range(2):
        pltpu.make_async_copy(bufs[b], agg_sp.at[sring[0].at[1]],
                              ssems[b]).wait()
    plsc.subcore_barrier()

    @pl.when(s < 15)
    def _():
        pltpu.sync_copy(agg_sp.at[pl.ds(s * 640, 640)],
                        out_hbm.at[c, pl.ds(s * 640, 640)])

    @pl.when(s == 15)
    def _():
        pltpu.sync_copy(agg_sp.at[pl.ds(9600, 400)],
                        out_hbm.at[c, pl.ds(9600, 400)])


def _scale_body(x_ref, dp_ref, y_ref, dis_ref):
    deg = dp_ref[0] + dp_ref[1] + 1.0
    dis = lax.rsqrt(deg)
    y_ref[...] = x_ref[...] * dis
    dis_ref[...] = dis


def _head_body(ap_ref, y_ref, dis_ref, w_ref, b_ref, o_ref):
    agg = (ap_ref[0] + ap_ref[1] + y_ref[...]) * dis_ref[...]
    for h in range(H):
        o_ref[h] = (
            jnp.dot(agg, w_ref[h], preferred_element_type=jnp.float32)
            + b_ref[h][None, :]
        )


def _sc_mesh():
    return plsc.VectorSubcoreMesh(core_axis_name="c", subcore_axis_name="s")


@jax.jit
def kernel(x, edge_index, W, b):
    deg_part = pl.kernel(
        _deg_body,
        out_type=jax.ShapeDtypeStruct((NC, 10, 1000), jnp.float32),
        mesh=_sc_mesh(),
        scratch_types=[
            pltpu.VMEM_SHARED((N,), jnp.float32),
            [pltpu.VMEM((2, CH), jnp.int32) for _ in range(4)],
            pltpu.VMEM((CH,), jnp.float32),
            pltpu.VMEM((1000,), jnp.float32),
            [pltpu.SemaphoreType.DMA for _ in range(4)],
            [pltpu.SemaphoreType.DMA for _ in range(2)],
        ],
    )(edge_index, jnp.ones((CH,), jnp.float32),
      jnp.zeros((1000,), jnp.float32))

    y, dis = pl.pallas_call(
        _scale_body,
        out_shape=(
            jax.ShapeDtypeStruct((N, D), jnp.float32),
            jax.ShapeDtypeStruct((N, 1), jnp.float32),
        ),
    )(x, deg_part.reshape(NC, N, 1))

    agg_part = pl.kernel(
        _agg_body,
        out_type=jax.ShapeDtypeStruct((NC, N, D), jnp.float32),
        mesh=_sc_mesh(),
        scratch_types=[
            pltpu.VMEM_SHARED((N, D), jnp.float32),
            [pltpu.VMEM((2, CH), jnp.int32) for _ in range(4)],
            [pltpu.VMEM((CH, D), jnp.float32) for _ in range(2)],
            [pltpu.SemaphoreType.DMA for _ in range(4)],
            [pltpu.SemaphoreType.DMA for _ in range(2)],
            [pltpu.SemaphoreType.DMA for _ in range(2)],
        ],
    )(y, edge_index, jnp.zeros((CH, D), jnp.float32))

    bm = 1000
    out4 = pl.pallas_call(
        _head_body,
        grid=(N // bm,),
        in_specs=[
            pl.BlockSpec((NC, bm, D), lambda i: (0, i, 0)),
            pl.BlockSpec((bm, D), lambda i: (i, 0)),
            pl.BlockSpec((bm, 1), lambda i: (i, 0)),
            pl.BlockSpec((H, D, D), lambda i: (0, 0, 0)),
            pl.BlockSpec((H, D), lambda i: (0, 0)),
        ],
        out_specs=pl.BlockSpec((H, bm, D), lambda i: (0, i, 0)),
        out_shape=jax.ShapeDtypeStruct((H, N, D), jnp.float32),
    )(agg_part, y, dis, W, b)

    return jnp.transpose(out4, (1, 2, 0))
